# baseline (speedup 1.0000x reference)
import jax
import jax.numpy as jnp
from jax import lax
from jax.experimental import pallas as pl
from jax.experimental.pallas import tpu as pltpu

H = 32
W = 32
HALF = 128
EMBED = 2 * HALF
P = H * W
BB = 4


def _pos_kernel(rows_ref, cols_ref, out_ref, scratch):
    @pl.when(pl.program_id(0) == 0)
    def _build():
        p_idx = lax.broadcasted_iota(jnp.int32, (W, P), 1)
        x_idx = lax.broadcasted_iota(jnp.int32, (W, P), 0)
        sel_c = (p_idx % W == x_idx).astype(jnp.float32)
        sel_r = (p_idx // W == x_idx).astype(jnp.float32)
        cols = cols_ref[0:W, :]
        rows = rows_ref[0:H, :]
        dn = (((0,), (0,)), ((), ()))
        top = lax.dot_general(cols, sel_c, dn,
                              preferred_element_type=jnp.float32)
        bot = lax.dot_general(rows, sel_r, dn,
                              preferred_element_type=jnp.float32)
        scratch[0:HALF, :] = top
        scratch[HALF:EMBED, :] = bot

    for ib in range(BB):
        out_ref[ib] = scratch[:]


def kernel(pixel_values, rows_emb, cols_emb):
    b = pixel_values.shape[0]
    out = pl.pallas_call(
        _pos_kernel,
        grid=(b // BB,),
        in_specs=[
            pl.BlockSpec(rows_emb.shape, lambda i: (0, 0)),
            pl.BlockSpec(cols_emb.shape, lambda i: (0, 0)),
        ],
        out_specs=pl.BlockSpec((BB, EMBED, P), lambda i: (i, 0, 0)),
        out_shape=jax.ShapeDtypeStruct((b, EMBED, P), jnp.float32),
        scratch_shapes=[pltpu.VMEM((EMBED, P), jnp.float32)],
    )(rows_emb, cols_emb)
    return out.reshape(b, EMBED, H, W)
